# Initial kernel scaffold; baseline (speedup 1.0000x reference)
import jax
import jax.numpy as jnp
from jax.experimental import pallas as pl


def kernel(logits, targets):
    raise NotImplementedError("write your pallas kernel here")


# speedup vs baseline: 2.1947x; 2.1947x over previous
import functools

import jax
import jax.numpy as jnp
import numpy as np
from jax.experimental import pallas as pl

_Z = np.int32(0)

B, S, V = 256, 128, 2048
EOS = 0
PAD = 2048
EPS = 1e-8


def _chamfer_kernel(lref, tref, label_ref, eos_ref):
    b = pl.program_id(0)

    l = lref[0]
    t = tref[0]

    M = jnp.max(l, axis=1, keepdims=True)
    E = jnp.exp(l - M)
    Z = jnp.sum(E, axis=1, keepdims=True)
    P0 = E[:, 0:1] / Z
    sumsq = jnp.sum(E * E, axis=1, keepdims=True) / (Z * Z)

    iota_v = jax.lax.broadcasted_iota(jnp.int32, (V, S), 0)
    Cf = (iota_v == t).astype(jnp.float32)
    valid_row = (t != EOS).astype(jnp.float32)
    OH = Cf * valid_row
    mq = jnp.sum(OH, axis=0, keepdims=True)

    ones_v = jnp.ones((V, 1), jnp.float32)
    e0_v = (jax.lax.broadcasted_iota(jnp.int32, (V, 1), 0) == 0).astype(jnp.float32)
    dn = (((0,), (0,)), ((), ()))
    has_col = jax.lax.dot_general(Cf, ones_v, dn, preferred_element_type=jnp.float32)
    pos_col = jax.lax.dot_general(Cf, e0_v, dn, preferred_element_type=jnp.float32)
    mp_col = has_col - pos_col

    dn2 = (((1,), (0,)), ((), ()))
    Graw = jax.lax.dot_general(E, OH, dn2, preferred_element_type=jnp.float32)
    G = Graw / Z

    x2 = mp_col * (sumsq - P0 * P0)
    d = x2 + mq - 2.0 * (mp_col * G)
    cham_x = jnp.min(d, axis=1)
    cham_y = jnp.min(d, axis=0)
    label_part = (jnp.sum(cham_x) + jnp.sum(cham_y)) * (1.0 / (B * S))

    tt = 1.0 - mp_col
    log_p = jnp.maximum(jnp.log(P0), -100.0)
    log_1mp = jnp.maximum(jnp.log(1.0 - P0), -100.0)
    bce = -(tt * log_p + (1.0 - tt) * log_1mp)
    n_pos = jnp.sum(pos_col)
    n_head = jnp.sum(mp_col)
    eos_part = (0.5 * jnp.sum(bce * pos_col) / (n_pos + EPS)
                + 0.5 * jnp.sum(bce * mp_col) / (n_head + EPS)) * (1.0 / B)

    @pl.when(b == 0)
    def _init():
        label_ref[...] = jnp.zeros((1, 1), jnp.float32)
        eos_ref[...] = jnp.zeros((1, 1), jnp.float32)

    label_ref[...] += jnp.broadcast_to(label_part, (1, 1))
    eos_ref[...] += jnp.broadcast_to(eos_part, (1, 1))


@jax.jit
def kernel(logits, targets):
    targets = targets.astype(jnp.int32).reshape(B, 1, S)
    label, eos = pl.pallas_call(
        _chamfer_kernel,
        grid=(B,),
        in_specs=[
            pl.BlockSpec((1, S, V), lambda b: (b, _Z, _Z)),
            pl.BlockSpec((1, 1, S), lambda b: (b, _Z, _Z)),
        ],
        out_specs=[
            pl.BlockSpec((1, 1), lambda b: (_Z, _Z)),
            pl.BlockSpec((1, 1), lambda b: (_Z, _Z)),
        ],
        out_shape=[
            jax.ShapeDtypeStruct((1, 1), jnp.float32),
            jax.ShapeDtypeStruct((1, 1), jnp.float32),
        ],
    )(logits, targets)
    return (label[0, 0], eos[0, 0])


# speedup vs baseline: 2.2187x; 1.0109x over previous
import functools

import jax
import jax.numpy as jnp
import numpy as np
from jax.experimental import pallas as pl

_Z = np.int32(0)

B, S, V = 256, 128, 2048
EOS = 0
PAD = 2048
EPS = 1e-8


def _chamfer_kernel(lref, tref, label_ref, eos_ref):
    b = pl.program_id(0)

    l = lref[0]
    t = tref[0]

    M = jnp.max(l, axis=1, keepdims=True)
    E = jnp.exp(l - M)
    Z = jnp.sum(E, axis=1, keepdims=True)
    P0 = E[:, 0:1] / Z
    sumsq = jnp.sum(E * E, axis=1, keepdims=True) / (Z * Z)

    iota_v = jax.lax.broadcasted_iota(jnp.int32, (V, S), 0)
    Cf = (iota_v == t).astype(jnp.bfloat16)
    valid_row = (t != EOS).astype(jnp.bfloat16)
    OH = Cf * valid_row
    mq = ((t != EOS) & (t != PAD)).astype(jnp.float32)

    ones_v = jnp.ones((V, 1), jnp.bfloat16)
    e0_v = (jax.lax.broadcasted_iota(jnp.int32, (V, 1), 0) == 0).astype(jnp.bfloat16)
    dn = (((0,), (0,)), ((), ()))
    has_col = jax.lax.dot_general(Cf, ones_v, dn, preferred_element_type=jnp.float32)
    pos_col = jax.lax.dot_general(Cf, e0_v, dn, preferred_element_type=jnp.float32)
    mp_col = has_col - pos_col

    dn2 = (((1,), (0,)), ((), ()))
    Graw = jax.lax.dot_general(E.astype(jnp.bfloat16), OH, dn2,
                               preferred_element_type=jnp.float32)
    G = Graw / Z

    x2 = mp_col * (sumsq - P0 * P0)
    d = x2 + mq - 2.0 * (mp_col * G)
    cham_x = jnp.min(d, axis=1)
    cham_y = jnp.min(d, axis=0)
    label_part = (jnp.sum(cham_x) + jnp.sum(cham_y)) * (1.0 / (B * S))

    tt = 1.0 - mp_col
    log_p = jnp.maximum(jnp.log(P0), -100.0)
    log_1mp = jnp.maximum(jnp.log(1.0 - P0), -100.0)
    bce = -(tt * log_p + (1.0 - tt) * log_1mp)
    n_pos = jnp.sum(pos_col)
    n_head = jnp.sum(mp_col)
    eos_part = (0.5 * jnp.sum(bce * pos_col) / (n_pos + EPS)
                + 0.5 * jnp.sum(bce * mp_col) / (n_head + EPS)) * (1.0 / B)

    @pl.when(b == 0)
    def _init():
        label_ref[...] = jnp.zeros((1, 1), jnp.float32)
        eos_ref[...] = jnp.zeros((1, 1), jnp.float32)

    label_ref[...] += jnp.broadcast_to(label_part, (1, 1))
    eos_ref[...] += jnp.broadcast_to(eos_part, (1, 1))


@jax.jit
def kernel(logits, targets):
    targets = targets.astype(jnp.int32).reshape(B, 1, S)
    label, eos = pl.pallas_call(
        _chamfer_kernel,
        grid=(B,),
        in_specs=[
            pl.BlockSpec((1, S, V), lambda b: (b, _Z, _Z)),
            pl.BlockSpec((1, 1, S), lambda b: (b, _Z, _Z)),
        ],
        out_specs=[
            pl.BlockSpec((1, 1), lambda b: (_Z, _Z)),
            pl.BlockSpec((1, 1), lambda b: (_Z, _Z)),
        ],
        out_shape=[
            jax.ShapeDtypeStruct((1, 1), jnp.float32),
            jax.ShapeDtypeStruct((1, 1), jnp.float32),
        ],
    )(logits, targets)
    return (label[0, 0], eos[0, 0])


# speedup vs baseline: 2.3631x; 1.0651x over previous
import functools

import jax
import jax.numpy as jnp
import numpy as np
from jax.experimental import pallas as pl

_Z = np.int32(0)

B, S, V = 256, 128, 2048
EOS = 0
PAD = 2048
EPS = 1e-8


def _chamfer_kernel(lref, tref, tcref, label_ref, eos_ref):
    b = pl.program_id(0)

    l = lref[0]
    t = tref[0]
    tc = tcref[0]

    M = jnp.max(l, axis=1, keepdims=True)
    E = jnp.exp(l - M)
    Z = jnp.sum(E, axis=1, keepdims=True)
    P0 = E[:, 0:1] / Z
    sumsq = jnp.sum(E * E, axis=1, keepdims=True) / (Z * Z)

    iota_v = jax.lax.broadcasted_iota(jnp.int32, (V, S), 0)
    Cf = (iota_v == t).astype(jnp.bfloat16)
    valid_row = (t != EOS).astype(jnp.bfloat16)
    OH = Cf * valid_row
    mq = ((t != EOS) & (t != PAD)).astype(jnp.float32)

    pos_col = (tc == EOS).astype(jnp.float32)
    mp_col = ((tc != EOS) & (tc != PAD)).astype(jnp.float32)

    dn2 = (((1,), (0,)), ((), ()))
    Graw = jax.lax.dot_general(E.astype(jnp.bfloat16), OH, dn2,
                               preferred_element_type=jnp.float32)
    G = Graw / Z

    x2 = mp_col * (sumsq - P0 * P0)
    d = x2 + mq - 2.0 * (mp_col * G)
    cham_x = jnp.min(d, axis=1)
    cham_y = jnp.min(d, axis=0)
    label_part = (jnp.sum(cham_x) + jnp.sum(cham_y)) * (1.0 / (B * S))

    tt = 1.0 - mp_col
    log_p = jnp.maximum(jnp.log(P0), -100.0)
    log_1mp = jnp.maximum(jnp.log(1.0 - P0), -100.0)
    bce = -(tt * log_p + (1.0 - tt) * log_1mp)
    n_pos = jnp.sum(pos_col)
    n_head = jnp.sum(mp_col)
    eos_part = (0.5 * jnp.sum(bce * pos_col) / (n_pos + EPS)
                + 0.5 * jnp.sum(bce * mp_col) / (n_head + EPS)) * (1.0 / B)

    @pl.when(b == 0)
    def _init():
        label_ref[...] = jnp.zeros((1, 1), jnp.float32)
        eos_ref[...] = jnp.zeros((1, 1), jnp.float32)

    label_ref[...] += jnp.broadcast_to(label_part, (1, 1))
    eos_ref[...] += jnp.broadcast_to(eos_part, (1, 1))


@jax.jit
def kernel(logits, targets):
    targets = targets.astype(jnp.int32)
    targets_row = targets.reshape(B, 1, S)
    targets_col = targets.reshape(B, S, 1)
    label, eos = pl.pallas_call(
        _chamfer_kernel,
        grid=(B,),
        in_specs=[
            pl.BlockSpec((1, S, V), lambda b: (b, _Z, _Z)),
            pl.BlockSpec((1, 1, S), lambda b: (b, _Z, _Z)),
            pl.BlockSpec((1, S, 1), lambda b: (b, _Z, _Z)),
        ],
        out_specs=[
            pl.BlockSpec((1, 1), lambda b: (_Z, _Z)),
            pl.BlockSpec((1, 1), lambda b: (_Z, _Z)),
        ],
        out_shape=[
            jax.ShapeDtypeStruct((1, 1), jnp.float32),
            jax.ShapeDtypeStruct((1, 1), jnp.float32),
        ],
    )(logits, targets_row, targets_col)
    return (label[0, 0], eos[0, 0])
